# baseline (speedup 1.0000x reference)
import functools

import jax
import jax.numpy as jnp
from jax.experimental import pallas as pl
from jax.experimental.pallas import tpu as pltpu


_TILE_H = 32

_dot = functools.partial(
    jax.lax.dot_general,
    dimension_numbers=(((1,), (0,)), ((), ())),
    preferred_element_type=jnp.float32)


def _puca_kernel(xm_ref, x0_ref, xp_ref, w2_ref, b2_ref, ws_ref,
                 u4a_ref, u4b_ref, bmid_ref, ftw_ref, bout_ref, o_ref):
    n = o_ref.shape[2]
    ht = _TILE_H
    w = n // ht
    r0 = pl.program_id(1) * ht

    xrefs = (xm_ref, x0_ref, xp_ref)
    taps = []
    for kh in range(3):
        for kw in range(3):
            if kh == 1 and kw == 1:
                continue
            taps.append(xrefs[kw][0, :, pl.ds((r0 + 4 + kh) * w, n)])
    xcol = jnp.concatenate(taps, axis=0)

    tcf = _dot(w2_ref[...], xcol) + b2_ref[...]
    skip = _dot(ws_ref[...], xcol)
    tcb = tcf.astype(jnp.bfloat16)

    blk = 2 * w
    y = jnp.concatenate(
        [jnp.concatenate([tcb[:, (2 * i + a) * blk:(2 * i + a + 1) * blk]
                          for i in range(ht // 4)], axis=1)
         for a in range(2)], axis=0)

    hw = w // 2
    nrow = (n // 2) // w
    y0 = jnp.concatenate([y[:, j * w:j * w + hw] for j in range(nrow)],
                         axis=1)
    y1 = jnp.concatenate([y[:, j * w + hw:(j + 1) * w] for j in range(nrow)],
                         axis=1)
    zs = _dot(u4a_ref[...], y0) + _dot(u4b_ref[...], y1)

    z0, z1 = zs[:256], zs[256:]
    pieces = []
    for j in range(nrow):
        pieces.append(z0[:, j * hw:(j + 1) * hw])
        pieces.append(z1[:, j * hw:(j + 1) * hw])
    mid = jnp.concatenate(pieces, axis=1)
    mid = (mid + bmid_ref[...]).astype(jnp.bfloat16)

    pieces = []
    for i in range(ht // 4):
        pieces.append(mid[0:128, i * blk:(i + 1) * blk])
        pieces.append(mid[128:256, i * blk:(i + 1) * blk])
    ymid = jnp.concatenate(pieces, axis=1)

    out = _dot(ftw_ref[...], ymid) + skip + bout_ref[...]
    o_ref[0] = out


def kernel(x, fused_mc_w, fused_mc_b, down0_w, down0_b, up0_w, up0_b,
           fused_tail_w, fused_tail_b):
    B, cimg, H, W = x.shape
    p, mp = 4, 1

    x16 = x.astype(jnp.bfloat16)
    xpad = jnp.pad(x16, ((0, 0), (0, 0), (p, p), (p, p)), mode='reflect')
    ones = jnp.ones((B, 1, H + 2 * p, W + 2 * p), jnp.bfloat16)
    xz = jnp.pad(jnp.concatenate([xpad, ones], axis=1),
                 ((0, 0), (0, 0), (mp, mp), (mp, mp)))
    ca, hz = cimg + 1, H + 2 * p + 2 * mp
    width = fused_mc_w.shape[0]
    w2c = down0_w.shape[0]

    def shifted(delta):
        s = xz[:, :, :, 5 + delta:5 + delta + W]
        s = s.reshape(B, ca, hz, W // 4, 4).transpose(0, 1, 2, 4, 3)
        return s.reshape(B, ca, hz * W)
    xm, x0, xp1 = shifted(-1), shifted(0), shifted(1)

    w2 = down0_w @ fused_mc_w
    b2 = down0_w @ fused_mc_b + down0_b
    ws = fused_tail_w @ fused_mc_w
    bout = fused_tail_w @ fused_mc_b + fused_tail_b
    u6 = up0_w.reshape(width, 2, 2, w2c, 2, 2)
    u4 = u6.transpose(2, 1, 0, 5, 4, 3).reshape(4 * width, 2, 2 * w2c)
    u4a, u4b = u4[:, 0, :], u4[:, 1, :]
    ub = up0_b.reshape(width, 2, 2).transpose(1, 0, 2).reshape(2 * width, 2)
    laneq = (jnp.arange(_TILE_H // 2 * W) % W) // (W // 2)
    bmid = ub[:, laneq]

    nt = _TILE_H * W
    out = pl.pallas_call(
        _puca_kernel,
        out_shape=jax.ShapeDtypeStruct((B, cimg, H * W), jnp.float32),
        grid=(B, H // _TILE_H),
        in_specs=[
            pl.BlockSpec((1, ca, hz * W), lambda b, t: (b, 0, 0)),
            pl.BlockSpec((1, ca, hz * W), lambda b, t: (b, 0, 0)),
            pl.BlockSpec((1, ca, hz * W), lambda b, t: (b, 0, 0)),
            pl.BlockSpec(w2.shape, lambda b, t: (0, 0)),
            pl.BlockSpec((w2c, 1), lambda b, t: (0, 0)),
            pl.BlockSpec(ws.shape, lambda b, t: (0, 0)),
            pl.BlockSpec(u4a.shape, lambda b, t: (0, 0)),
            pl.BlockSpec(u4b.shape, lambda b, t: (0, 0)),
            pl.BlockSpec(bmid.shape, lambda b, t: (0, 0)),
            pl.BlockSpec(fused_tail_w.shape, lambda b, t: (0, 0)),
            pl.BlockSpec((cimg, 1), lambda b, t: (0, 0)),
        ],
        out_specs=pl.BlockSpec((1, cimg, nt), lambda b, t: (b, 0, t)),
        compiler_params=pltpu.CompilerParams(
            dimension_semantics=("parallel", "arbitrary")),
    )(xm, x0, xp1, w2.astype(jnp.bfloat16), b2.reshape(w2c, 1),
      ws.astype(jnp.bfloat16), u4a.astype(jnp.bfloat16),
      u4b.astype(jnp.bfloat16), bmid,
      fused_tail_w.astype(jnp.bfloat16), bout.reshape(cimg, 1))

    out = out.reshape(B, cimg, H, 4, W // 4).transpose(0, 1, 2, 4, 3)
    return out.reshape(B, cimg, H, W)


# speedup vs baseline: 10.8663x; 10.8663x over previous
import functools

import jax
import jax.numpy as jnp
from jax.experimental import pallas as pl
from jax.experimental.pallas import tpu as pltpu


_TILE_H = 32

_dot = functools.partial(
    jax.lax.dot_general,
    dimension_numbers=(((1,), (0,)), ((), ())),
    preferred_element_type=jnp.float32)


def _puca_kernel(xz_ref, w2_ref, b2_ref, ws_ref, u4a_ref, u4b_ref,
                 bmid_ref, ftw_ref, bout_ref, pm_ref, pt_ref, o_ref):
    ht = o_ref.shape[2]
    wo = o_ref.shape[3]
    n = ht * wo
    r0 = pl.program_id(1) * ht

    rows_all = xz_ref[0, :, pl.ds(r0, ht + 8), :]
    taps = []
    for kh in range(3):
        for kw in range(3):
            if kh == 1 and kw == 1:
                continue
            taps.append(rows_all[:, 4 + kh:4 + kh + ht, 4 + kw:4 + kw + wo])
    xcol = jnp.concatenate(taps, axis=0)
    xcol = jax.lax.dot_general(xcol, pm_ref[...], (((2,), (0,)), ((), ())),
                               preferred_element_type=jnp.float32)
    xcol = xcol.astype(jnp.bfloat16)
    xcol = xcol.reshape(8 * rows_all.shape[0], n)

    tcf = _dot(w2_ref[...], xcol) + b2_ref[...]
    skip = _dot(ws_ref[...], xcol)
    tcb = tcf.astype(jnp.bfloat16)

    blk = 2 * wo
    y = jnp.concatenate(
        [jnp.concatenate([tcb[:, (2 * i + a) * blk:(2 * i + a + 1) * blk]
                          for i in range(ht // 4)], axis=1)
         for a in range(2)], axis=0)

    hw = wo // 2
    nrow = (n // 2) // wo
    y0 = jnp.concatenate([y[:, j * wo:j * wo + hw] for j in range(nrow)],
                         axis=1)
    y1 = jnp.concatenate([y[:, j * wo + hw:(j + 1) * wo]
                          for j in range(nrow)], axis=1)
    zs = _dot(u4a_ref[...], y0) + _dot(u4b_ref[...], y1)

    z0, z1 = zs[:256], zs[256:]
    qp = []
    for j in range(nrow):
        qp.append(z0[:, j * hw:(j + 1) * hw])
        qp.append(z1[:, j * hw:(j + 1) * hw])
    mid = jnp.concatenate(qp, axis=1)
    mid = (mid + bmid_ref[...]).astype(jnp.bfloat16)

    pieces = []
    for i in range(ht // 4):
        pieces.append(mid[0:128, i * blk:(i + 1) * blk])
        pieces.append(mid[128:256, i * blk:(i + 1) * blk])
    ymid = jnp.concatenate(pieces, axis=1)

    out = _dot(ftw_ref[...], ymid) + skip + bout_ref[...]
    out = out.reshape(out.shape[0], ht, wo)
    o_ref[0] = jax.lax.dot_general(out, pt_ref[...], (((2,), (0,)), ((), ())),
                                   preferred_element_type=jnp.float32)


def kernel(x, fused_mc_w, fused_mc_b, down0_w, down0_b, up0_w, up0_b,
           fused_tail_w, fused_tail_b):
    B, cimg, H, W = x.shape
    p, mp = 4, 1

    x16 = x.astype(jnp.bfloat16)
    xp = jnp.pad(x16, ((0, 0), (0, 0), (p, p), (p, p)), mode='reflect')
    ones = jnp.ones((B, 1, H + 2 * p, W + 2 * p), jnp.bfloat16)
    xz = jnp.pad(jnp.concatenate([xp, ones], axis=1),
                 ((0, 0), (0, 0), (mp, mp), (mp, mp)))
    ca, hz, wz = cimg + 1, H + 2 * p + 2 * mp, W + 2 * p + 2 * mp
    width = fused_mc_w.shape[0]
    w2c = down0_w.shape[0]

    w2 = down0_w @ fused_mc_w
    b2 = down0_w @ fused_mc_b + down0_b
    ws = fused_tail_w @ fused_mc_w
    bout = fused_tail_w @ fused_mc_b + fused_tail_b
    u6 = up0_w.reshape(width, 2, 2, w2c, 2, 2)
    u4 = u6.transpose(2, 1, 0, 5, 4, 3).reshape(4 * width, 2, 2 * w2c)
    u4a, u4b = u4[:, 0, :], u4[:, 1, :]
    ub = up0_b.reshape(width, 2, 2).transpose(1, 0, 2).reshape(2 * width, 2)
    laneq = (jnp.arange(_TILE_H // 2 * W) % W) // (W // 2)
    bmid = ub[:, laneq]
    wi = jnp.arange(W)
    pm = jnp.zeros((W, W), jnp.float32).at[wi, (W // 4) * (wi % 4) + wi // 4].set(1.0)
    pt = pm.T

    return pl.pallas_call(
        _puca_kernel,
        out_shape=jax.ShapeDtypeStruct((B, cimg, H, W), jnp.float32),
        grid=(B, H // _TILE_H),
        in_specs=[
            pl.BlockSpec((1, ca, hz, wz), lambda b, t: (b, 0, 0, 0)),
            pl.BlockSpec(w2.shape, lambda b, t: (0, 0)),
            pl.BlockSpec((w2c, 1), lambda b, t: (0, 0)),
            pl.BlockSpec(ws.shape, lambda b, t: (0, 0)),
            pl.BlockSpec(u4a.shape, lambda b, t: (0, 0)),
            pl.BlockSpec(u4b.shape, lambda b, t: (0, 0)),
            pl.BlockSpec(bmid.shape, lambda b, t: (0, 0)),
            pl.BlockSpec(fused_tail_w.shape, lambda b, t: (0, 0)),
            pl.BlockSpec((cimg, 1), lambda b, t: (0, 0)),
            pl.BlockSpec((W, W), lambda b, t: (0, 0)),
            pl.BlockSpec((W, W), lambda b, t: (0, 0)),
        ],
        out_specs=pl.BlockSpec((1, cimg, _TILE_H, W), lambda b, t: (b, 0, t, 0)),
        compiler_params=pltpu.CompilerParams(
            dimension_semantics=("parallel", "arbitrary")),
    )(xz, w2.astype(jnp.bfloat16), b2.reshape(w2c, 1),
      ws.astype(jnp.bfloat16), u4a.astype(jnp.bfloat16),
      u4b.astype(jnp.bfloat16), bmid,
      fused_tail_w.astype(jnp.bfloat16), bout.reshape(cimg, 1),
      pm.astype(jnp.bfloat16), pt)


# speedup vs baseline: 10.9865x; 1.0111x over previous
import functools

import jax
import jax.numpy as jnp
from jax.experimental import pallas as pl
from jax.experimental.pallas import tpu as pltpu


_TILE_H = 32

_dot = functools.partial(
    jax.lax.dot_general,
    dimension_numbers=(((1,), (0,)), ((), ())),
    preferred_element_type=jnp.float32)


def _puca_kernel(xz_ref, w2_ref, b2_ref, ws_ref, u4a_ref, u4b_ref,
                 bmid_ref, ftw_ref, bout_ref, pm_ref, pt_ref, o_ref):
    ht = o_ref.shape[2]
    wo = o_ref.shape[3]
    n = ht * wo
    r0 = pl.program_id(1) * ht

    rows_all = xz_ref[0, :, pl.ds(r0, ht + 8), :]
    taps = []
    for kh in range(3):
        for kw in range(3):
            if kh == 1 and kw == 1:
                continue
            taps.append(rows_all[:, 4 + kh:4 + kh + ht, 4 + kw:4 + kw + wo])
    xcol = jnp.concatenate(taps, axis=0)
    xcol = jax.lax.dot_general(xcol, pm_ref[...], (((2,), (0,)), ((), ())),
                               preferred_element_type=jnp.float32)
    xcol = xcol.astype(jnp.bfloat16)
    xcol = xcol.reshape(8 * rows_all.shape[0], n)

    tcf = _dot(w2_ref[...], xcol) + b2_ref[...]
    skip = _dot(ws_ref[...], xcol)
    tcb = tcf.astype(jnp.bfloat16)

    blk = 2 * wo
    y = jnp.concatenate(
        [jnp.concatenate([tcb[:, (2 * i + a) * blk:(2 * i + a + 1) * blk]
                          for i in range(ht // 4)], axis=1)
         for a in range(2)], axis=0)

    hw = wo // 2
    nrow = (n // 2) // wo
    y0 = jnp.concatenate([y[:, j * wo:j * wo + hw] for j in range(nrow)],
                         axis=1)
    y1 = jnp.concatenate([y[:, j * wo + hw:(j + 1) * wo]
                          for j in range(nrow)], axis=1)
    zs = _dot(u4a_ref[...], y0) + _dot(u4b_ref[...], y1)

    z0, z1 = zs[:256], zs[256:]
    qp = []
    for j in range(nrow):
        qp.append(z0[:, j * hw:(j + 1) * hw])
        qp.append(z1[:, j * hw:(j + 1) * hw])
    mid = jnp.concatenate(qp, axis=1)
    mid = (mid + bmid_ref[...]).astype(jnp.bfloat16)

    pieces = []
    for i in range(ht // 4):
        pieces.append(mid[0:128, i * blk:(i + 1) * blk])
        pieces.append(mid[128:256, i * blk:(i + 1) * blk])
    ymid = jnp.concatenate(pieces, axis=1)

    out = _dot(ftw_ref[...], ymid) + skip + bout_ref[...]
    out = out.reshape(out.shape[0], ht, wo)
    o_ref[0] = jax.lax.dot_general(out, pt_ref[...], (((2,), (0,)), ((), ())),
                                   preferred_element_type=jnp.float32)


def kernel(x, fused_mc_w, fused_mc_b, down0_w, down0_b, up0_w, up0_b,
           fused_tail_w, fused_tail_b):
    B, cimg, H, W = x.shape
    p, mp = 4, 1

    x16 = x.astype(jnp.bfloat16)
    xp = jnp.pad(x16, ((0, 0), (0, 0), (p, p), (p, p)), mode='reflect')
    ones = jnp.ones((B, 1, H + 2 * p, W + 2 * p), jnp.bfloat16)
    xz = jnp.pad(jnp.concatenate([xp, ones], axis=1),
                 ((0, 0), (0, 0), (mp, mp), (mp, mp)))
    ca, hz, wz = cimg + 1, H + 2 * p + 2 * mp, W + 2 * p + 2 * mp
    width = fused_mc_w.shape[0]
    w2c = down0_w.shape[0]

    w2 = down0_w @ fused_mc_w
    b2 = down0_w @ fused_mc_b + down0_b
    ws = fused_tail_w @ fused_mc_w
    bout = fused_tail_w @ fused_mc_b + fused_tail_b
    u6 = up0_w.reshape(width, 2, 2, w2c, 2, 2)
    u4 = u6.transpose(2, 1, 0, 5, 4, 3).reshape(4 * width, 2, 2 * w2c)
    u4a, u4b = u4[:, 0, :], u4[:, 1, :]
    ub = up0_b.reshape(width, 2, 2).transpose(1, 0, 2).reshape(2 * width, 2)
    bm_row = jnp.repeat(ub, W // 2, axis=1)
    bmid = jnp.tile(bm_row, (1, _TILE_H // 2))
    wi = jnp.arange(W)
    pm = jnp.zeros((W, W), jnp.float32).at[wi, (W // 4) * (wi % 4) + wi // 4].set(1.0)
    pt = pm.T

    return pl.pallas_call(
        _puca_kernel,
        out_shape=jax.ShapeDtypeStruct((B, cimg, H, W), jnp.float32),
        grid=(B, H // _TILE_H),
        in_specs=[
            pl.BlockSpec((1, ca, hz, wz), lambda b, t: (b, 0, 0, 0)),
            pl.BlockSpec(w2.shape, lambda b, t: (0, 0)),
            pl.BlockSpec((w2c, 1), lambda b, t: (0, 0)),
            pl.BlockSpec(ws.shape, lambda b, t: (0, 0)),
            pl.BlockSpec(u4a.shape, lambda b, t: (0, 0)),
            pl.BlockSpec(u4b.shape, lambda b, t: (0, 0)),
            pl.BlockSpec(bmid.shape, lambda b, t: (0, 0)),
            pl.BlockSpec(fused_tail_w.shape, lambda b, t: (0, 0)),
            pl.BlockSpec((cimg, 1), lambda b, t: (0, 0)),
            pl.BlockSpec((W, W), lambda b, t: (0, 0)),
            pl.BlockSpec((W, W), lambda b, t: (0, 0)),
        ],
        out_specs=pl.BlockSpec((1, cimg, _TILE_H, W), lambda b, t: (b, 0, t, 0)),
        compiler_params=pltpu.CompilerParams(
            dimension_semantics=("parallel", "arbitrary")),
    )(xz, w2.astype(jnp.bfloat16), b2.reshape(w2c, 1),
      ws.astype(jnp.bfloat16), u4a.astype(jnp.bfloat16),
      u4b.astype(jnp.bfloat16), bmid,
      fused_tail_w.astype(jnp.bfloat16), bout.reshape(cimg, 1),
      pm.astype(jnp.bfloat16), pt)


# speedup vs baseline: 11.4391x; 1.0412x over previous
import functools

import jax
import jax.numpy as jnp
from jax.experimental import pallas as pl
from jax.experimental.pallas import tpu as pltpu


_TILE_H = 64

_dot = functools.partial(
    jax.lax.dot_general,
    dimension_numbers=(((1,), (0,)), ((), ())),
    preferred_element_type=jnp.float32)


def _puca_kernel(xz_ref, w2_ref, b2_ref, ws_ref, u4a_ref, u4b_ref,
                 bmid_ref, ftw_ref, bout_ref, pm_ref, pt_ref, o_ref):
    ht = o_ref.shape[2]
    wo = o_ref.shape[3]
    n = ht * wo
    r0 = pl.program_id(1) * ht

    rows_all = xz_ref[0, :, pl.ds(r0, ht + 8), :]
    taps = []
    for kh in range(3):
        for kw in range(3):
            if kh == 1 and kw == 1:
                continue
            taps.append(rows_all[:, 4 + kh:4 + kh + ht, 4 + kw:4 + kw + wo])
    xcol = jnp.concatenate(taps, axis=0)
    xcol = jax.lax.dot_general(xcol, pm_ref[...], (((2,), (0,)), ((), ())),
                               preferred_element_type=jnp.float32)
    xcol = xcol.astype(jnp.bfloat16)
    xcol = xcol.reshape(8 * rows_all.shape[0], n)

    tcf = _dot(w2_ref[...], xcol) + b2_ref[...]
    skip = _dot(ws_ref[...], xcol)
    tcb = tcf.astype(jnp.bfloat16)

    blk = 2 * wo
    y = jnp.concatenate(
        [jnp.concatenate([tcb[:, (2 * i + a) * blk:(2 * i + a + 1) * blk]
                          for i in range(ht // 4)], axis=1)
         for a in range(2)], axis=0)

    hw = wo // 2
    nrow = (n // 2) // wo
    y0 = jnp.concatenate([y[:, j * wo:j * wo + hw] for j in range(nrow)],
                         axis=1)
    y1 = jnp.concatenate([y[:, j * wo + hw:(j + 1) * wo]
                          for j in range(nrow)], axis=1)
    zs = _dot(u4a_ref[...], y0) + _dot(u4b_ref[...], y1)

    z0, z1 = zs[:256], zs[256:]
    qp = []
    for j in range(nrow):
        qp.append(z0[:, j * hw:(j + 1) * hw])
        qp.append(z1[:, j * hw:(j + 1) * hw])
    mid = jnp.concatenate(qp, axis=1)
    mid = (mid + bmid_ref[...]).astype(jnp.bfloat16)

    pieces = []
    for i in range(ht // 4):
        pieces.append(mid[0:128, i * blk:(i + 1) * blk])
        pieces.append(mid[128:256, i * blk:(i + 1) * blk])
    ymid = jnp.concatenate(pieces, axis=1)

    out = _dot(ftw_ref[...], ymid) + skip + bout_ref[...]
    out = out.reshape(out.shape[0], ht, wo)
    o_ref[0] = jax.lax.dot_general(out, pt_ref[...], (((2,), (0,)), ((), ())),
                                   preferred_element_type=jnp.float32)


def kernel(x, fused_mc_w, fused_mc_b, down0_w, down0_b, up0_w, up0_b,
           fused_tail_w, fused_tail_b):
    B, cimg, H, W = x.shape
    p, mp = 4, 1

    x16 = x.astype(jnp.bfloat16)
    xp = jnp.pad(x16, ((0, 0), (0, 0), (p, p), (p, p)), mode='reflect')
    ones = jnp.ones((B, 1, H + 2 * p, W + 2 * p), jnp.bfloat16)
    xz = jnp.pad(jnp.concatenate([xp, ones], axis=1),
                 ((0, 0), (0, 0), (mp, mp), (mp, mp)))
    ca, hz, wz = cimg + 1, H + 2 * p + 2 * mp, W + 2 * p + 2 * mp
    width = fused_mc_w.shape[0]
    w2c = down0_w.shape[0]

    w2 = down0_w @ fused_mc_w
    b2 = down0_w @ fused_mc_b + down0_b
    ws = fused_tail_w @ fused_mc_w
    bout = fused_tail_w @ fused_mc_b + fused_tail_b
    u6 = up0_w.reshape(width, 2, 2, w2c, 2, 2)
    u4 = u6.transpose(2, 1, 0, 5, 4, 3).reshape(4 * width, 2, 2 * w2c)
    u4a, u4b = u4[:, 0, :], u4[:, 1, :]
    ub = up0_b.reshape(width, 2, 2).transpose(1, 0, 2).reshape(2 * width, 2)
    bm_row = jnp.repeat(ub, W // 2, axis=1)
    bmid = jnp.tile(bm_row, (1, _TILE_H // 2))
    wi = jnp.arange(W)
    pm = jnp.zeros((W, W), jnp.float32).at[wi, (W // 4) * (wi % 4) + wi // 4].set(1.0)
    pt = pm.T

    return pl.pallas_call(
        _puca_kernel,
        out_shape=jax.ShapeDtypeStruct((B, cimg, H, W), jnp.float32),
        grid=(B, H // _TILE_H),
        in_specs=[
            pl.BlockSpec((1, ca, hz, wz), lambda b, t: (b, 0, 0, 0)),
            pl.BlockSpec(w2.shape, lambda b, t: (0, 0)),
            pl.BlockSpec((w2c, 1), lambda b, t: (0, 0)),
            pl.BlockSpec(ws.shape, lambda b, t: (0, 0)),
            pl.BlockSpec(u4a.shape, lambda b, t: (0, 0)),
            pl.BlockSpec(u4b.shape, lambda b, t: (0, 0)),
            pl.BlockSpec(bmid.shape, lambda b, t: (0, 0)),
            pl.BlockSpec(fused_tail_w.shape, lambda b, t: (0, 0)),
            pl.BlockSpec((cimg, 1), lambda b, t: (0, 0)),
            pl.BlockSpec((W, W), lambda b, t: (0, 0)),
            pl.BlockSpec((W, W), lambda b, t: (0, 0)),
        ],
        out_specs=pl.BlockSpec((1, cimg, _TILE_H, W), lambda b, t: (b, 0, t, 0)),
        compiler_params=pltpu.CompilerParams(
            dimension_semantics=("parallel", "arbitrary")),
    )(xz, w2.astype(jnp.bfloat16), b2.reshape(w2c, 1),
      ws.astype(jnp.bfloat16), u4a.astype(jnp.bfloat16),
      u4b.astype(jnp.bfloat16), bmid,
      fused_tail_w.astype(jnp.bfloat16), bout.reshape(cimg, 1),
      pm.astype(jnp.bfloat16), pt)
